# baseline (device time: 192526 ns/iter reference)
import jax
import jax.numpy as jnp
from jax import lax
from jax.experimental import pallas as pl
from jax.experimental.pallas import tpu as pltpu

T = 4096
V_SHARD = 8192
D = 2048
HALF = T // 2
K = 8
CH = HALF // K
NSEM = 64
WAVES = CH // NSEM


def kernel(ids, E):
    ids2d = ids[:, None]

    def body(ids_s, idv_ref, e_ref, out_ref, gstage, recv1, gsem, s1, r1, s2, r2):
        x = lax.axis_index("x")
        yy = lax.axis_index("y")
        base = x * HALF
        vlo = yy * V_SHARD

        barrier_sem = pltpu.get_barrier_semaphore()
        pl.semaphore_signal(barrier_sem, inc=1, device_id=(x, 1 - yy),
                            device_id_type=pl.DeviceIdType.MESH)
        pl.semaphore_signal(barrier_sem, inc=1, device_id=(1 - x, yy),
                            device_id_type=pl.DeviceIdType.MESH)
        pl.semaphore_wait(barrier_sem, 2)

        def row_copy(lidx, dst_row, k):
            return pltpu.make_async_copy(
                e_ref.at[pl.ds(lidx, 1), :],
                gstage.at[pl.ds(dst_row, 1), :],
                gsem.at[k],
            )

        def issue_row(t, dst_row, k):
            lidx = jnp.clip(ids_s[t] - vlo, 0, V_SHARD - 1)
            row_copy(lidx, dst_row, k).start()

        rdma1 = []
        rdma2 = []
        for c in range(K):
            lo = base + c * CH
            for k in range(NSEM):
                issue_row(lo + k, k, k)

            def wave(j, _, lo=lo):
                for k in range(NSEM):
                    row_copy(0, 0, k).wait()
                    issue_row(lo + j * NSEM + k, j * NSEM + k, k)
                return 0

            lax.fori_loop(1, WAVES, wave, 0)
            for k in range(NSEM):
                row_copy(0, 0, k).wait()

            out_ref[pl.ds(lo, CH), :] = gstage[:, :].astype(jnp.bfloat16)
            d1 = pltpu.make_async_remote_copy(
                src_ref=out_ref.at[pl.ds(lo, CH), :],
                dst_ref=recv1.at[pl.ds(c * CH, CH), :],
                send_sem=s1.at[c],
                recv_sem=r1.at[c],
                device_id=(x, 1 - yy),
                device_id_type=pl.DeviceIdType.MESH,
            )
            d1.start()
            rdma1.append(d1)

        for c in range(K):
            lo = base + c * CH
            rdma1[c].wait_recv()
            rdma1[c].wait_send()
            sel = (idv_ref[pl.ds(lo, CH), :] >= vlo) & (
                idv_ref[pl.ds(lo, CH), :] < vlo + V_SHARD)
            out_ref[pl.ds(lo, CH), :] = jnp.where(
                sel, out_ref[pl.ds(lo, CH), :], recv1[pl.ds(c * CH, CH), :])
            d2 = pltpu.make_async_remote_copy(
                src_ref=out_ref.at[pl.ds(lo, CH), :],
                dst_ref=out_ref.at[pl.ds(lo, CH), :],
                send_sem=s2.at[c],
                recv_sem=r2.at[c],
                device_id=(1 - x, yy),
                device_id_type=pl.DeviceIdType.MESH,
            )
            d2.start()
            rdma2.append(d2)

        for c in range(K):
            rdma2[c].wait_recv()
        for c in range(K):
            rdma2[c].wait_send()

    out = pl.pallas_call(
        body,
        out_shape=jax.ShapeDtypeStruct((T, D), jnp.bfloat16),
        in_specs=[
            pl.BlockSpec(memory_space=pltpu.SMEM),
            pl.BlockSpec(memory_space=pltpu.VMEM),
            pl.BlockSpec(memory_space=pl.ANY),
        ],
        out_specs=pl.BlockSpec(memory_space=pltpu.VMEM),
        scratch_shapes=[
            pltpu.VMEM((CH, D), jnp.float32),
            pltpu.VMEM((HALF, D), jnp.bfloat16),
            pltpu.SemaphoreType.DMA((NSEM,)),
            pltpu.SemaphoreType.DMA((K,)),
            pltpu.SemaphoreType.DMA((K,)),
            pltpu.SemaphoreType.DMA((K,)),
            pltpu.SemaphoreType.DMA((K,)),
        ],
        compiler_params=pltpu.CompilerParams(collective_id=0),
    )(ids, ids2d, E)
    return out.astype(jnp.float32)


# device time: 150209 ns/iter; 1.2817x vs baseline; 1.2817x over previous
import jax
import jax.numpy as jnp
from jax import lax
from jax.experimental import pallas as pl
from jax.experimental.pallas import tpu as pltpu

T = 4096
V_SHARD = 8192
D = 2048
HALF = T // 2
K = 8
CH = HALF // K


def kernel(ids, E):
    x = lax.axis_index("x")
    y = lax.axis_index("y")
    seg = lax.dynamic_slice(ids, (x * HALF,), (HALF,))
    lseg = seg - y * V_SHARD
    inr = (lseg >= 0) & (lseg < V_SHARD)
    order = jnp.argsort(~inr, stable=True).astype(jnp.int32)
    rows = jnp.clip(lseg[order], 0, V_SHARD - 1).astype(jnp.int32)
    cum = jnp.cumsum(
        inr.reshape(K, CH).sum(axis=1, dtype=jnp.int32)).astype(jnp.int32)
    ids2d = ids[:, None]

    def body(rows_s, pos_s, cum_s, idv_ref, e_ref, out_ref,
             gstage, recv1, gsem, s1, r1, s2, r2):
        xx = lax.axis_index("x")
        yy = lax.axis_index("y")
        base = xx * HALF
        vlo = yy * V_SHARD

        barrier_sem = pltpu.get_barrier_semaphore()
        pl.semaphore_signal(barrier_sem, inc=1, device_id=(xx, 1 - yy),
                            device_id_type=pl.DeviceIdType.MESH)
        pl.semaphore_signal(barrier_sem, inc=1, device_id=(1 - xx, yy),
                            device_id_type=pl.DeviceIdType.MESH)
        pl.semaphore_wait(barrier_sem, 2)

        rdma1 = []
        rdma2 = []
        for c in range(K):
            lo = base + c * CH
            start = cum_s[c - 1] if c > 0 else 0
            end = cum_s[c]

            def issue(j, _, off=c * CH):
                pltpu.make_async_copy(
                    e_ref.at[pl.ds(rows_s[j], 1), :],
                    gstage.at[pl.ds(pos_s[j] - off, 1), :],
                    gsem,
                ).start()
                return 0

            def drain(j, _):
                pltpu.make_async_copy(
                    e_ref.at[pl.ds(0, 1), :],
                    gstage.at[pl.ds(0, 1), :],
                    gsem,
                ).wait()
                return 0

            lax.fori_loop(start, end, issue, 0)
            lax.fori_loop(start, end, drain, 0)

            out_ref[pl.ds(lo, CH), :] = gstage[:, :].astype(jnp.bfloat16)
            d1 = pltpu.make_async_remote_copy(
                src_ref=out_ref.at[pl.ds(lo, CH), :],
                dst_ref=recv1.at[pl.ds(c * CH, CH), :],
                send_sem=s1.at[c],
                recv_sem=r1.at[c],
                device_id=(xx, 1 - yy),
                device_id_type=pl.DeviceIdType.MESH,
            )
            d1.start()
            rdma1.append(d1)

        for c in range(K):
            lo = base + c * CH
            rdma1[c].wait_recv()
            rdma1[c].wait_send()
            sel = (idv_ref[pl.ds(lo, CH), :] >= vlo) & (
                idv_ref[pl.ds(lo, CH), :] < vlo + V_SHARD)
            out_ref[pl.ds(lo, CH), :] = jnp.where(
                sel, out_ref[pl.ds(lo, CH), :], recv1[pl.ds(c * CH, CH), :])
            d2 = pltpu.make_async_remote_copy(
                src_ref=out_ref.at[pl.ds(lo, CH), :],
                dst_ref=out_ref.at[pl.ds(lo, CH), :],
                send_sem=s2.at[c],
                recv_sem=r2.at[c],
                device_id=(1 - xx, yy),
                device_id_type=pl.DeviceIdType.MESH,
            )
            d2.start()
            rdma2.append(d2)

        for c in range(K):
            rdma2[c].wait_recv()
        for c in range(K):
            rdma2[c].wait_send()

    out = pl.pallas_call(
        body,
        out_shape=jax.ShapeDtypeStruct((T, D), jnp.bfloat16),
        in_specs=[
            pl.BlockSpec(memory_space=pltpu.SMEM),
            pl.BlockSpec(memory_space=pltpu.SMEM),
            pl.BlockSpec(memory_space=pltpu.SMEM),
            pl.BlockSpec(memory_space=pltpu.VMEM),
            pl.BlockSpec(memory_space=pl.ANY),
        ],
        out_specs=pl.BlockSpec(memory_space=pltpu.VMEM),
        scratch_shapes=[
            pltpu.VMEM((CH, D), jnp.float32),
            pltpu.VMEM((HALF, D), jnp.bfloat16),
            pltpu.SemaphoreType.DMA,
            pltpu.SemaphoreType.DMA((K,)),
            pltpu.SemaphoreType.DMA((K,)),
            pltpu.SemaphoreType.DMA((K,)),
            pltpu.SemaphoreType.DMA((K,)),
        ],
        compiler_params=pltpu.CompilerParams(collective_id=0),
    )(rows, order, cum, ids2d, E)
    return out.astype(jnp.float32)


# device time: 144436 ns/iter; 1.3330x vs baseline; 1.0400x over previous
import jax
import jax.numpy as jnp
from jax import lax
from jax.experimental import pallas as pl
from jax.experimental.pallas import tpu as pltpu

T = 4096
V_SHARD = 8192
D = 2048
HALF = T // 2
K = 8
CH = HALF // K


def kernel(ids, E):
    x = lax.axis_index("x")
    y = lax.axis_index("y")
    seg = lax.dynamic_slice(ids, (x * HALF,), (HALF,))
    lseg = seg - y * V_SHARD
    inr = (lseg >= 0) & (lseg < V_SHARD)
    order = jnp.argsort(~inr, stable=True).astype(jnp.int32)
    rows = jnp.clip(lseg[order], 0, V_SHARD - 1).astype(jnp.int32)
    cum = jnp.cumsum(
        inr.reshape(K, CH).sum(axis=1, dtype=jnp.int32)).astype(jnp.int32)
    ids2d = ids[:, None]

    def body(rows_s, pos_s, cum_s, idv_ref, e_ref, out_ref,
             gstage, recv1, gsem, s1, r1, s2, r2):
        xx = lax.axis_index("x")
        yy = lax.axis_index("y")
        base = xx * HALF
        vlo = yy * V_SHARD

        barrier_sem = pltpu.get_barrier_semaphore()
        pl.semaphore_signal(barrier_sem, inc=1, device_id=(xx, 1 - yy),
                            device_id_type=pl.DeviceIdType.MESH)
        pl.semaphore_signal(barrier_sem, inc=1, device_id=(1 - xx, yy),
                            device_id_type=pl.DeviceIdType.MESH)
        pl.semaphore_wait(barrier_sem, 2)

        rdma1 = []
        rdma2 = []
        for c in range(K):
            lo = base + c * CH
            start = cum_s[c - 1] if c > 0 else 0
            end = cum_s[c]

            def issue(j, _, off=c * CH):
                pltpu.make_async_copy(
                    e_ref.at[pl.ds(rows_s[j], 1), :],
                    gstage.at[pl.ds(pos_s[j] - off, 1), :],
                    gsem,
                ).start()
                return 0

            def drain(j, _):
                pltpu.make_async_copy(
                    e_ref.at[pl.ds(0, 1), :],
                    gstage.at[pl.ds(0, 1), :],
                    gsem,
                ).wait()
                return 0

            lax.fori_loop(start, end, issue, 0)
            lax.fori_loop(start, end, drain, 0)

            out_ref[pl.ds(lo, CH), :] = gstage[:, :].astype(jnp.bfloat16)
            d1 = pltpu.make_async_remote_copy(
                src_ref=out_ref.at[pl.ds(lo, CH), :],
                dst_ref=recv1.at[pl.ds(c * CH, CH), :],
                send_sem=s1.at[c],
                recv_sem=r1.at[c],
                device_id=(xx, 1 - yy),
                device_id_type=pl.DeviceIdType.MESH,
            )
            d1.start()
            rdma1.append(d1)

        for c in range(K):
            lo = base + c * CH
            rdma1[c].wait_recv()
            rdma1[c].wait_send()
            sel = (idv_ref[pl.ds(lo, CH), :] >= vlo) & (
                idv_ref[pl.ds(lo, CH), :] < vlo + V_SHARD)
            out_ref[pl.ds(lo, CH), :] = jnp.where(
                sel, out_ref[pl.ds(lo, CH), :], recv1[pl.ds(c * CH, CH), :])
            d2 = pltpu.make_async_remote_copy(
                src_ref=out_ref.at[pl.ds(lo, CH), :],
                dst_ref=out_ref.at[pl.ds(lo, CH), :],
                send_sem=s2.at[c],
                recv_sem=r2.at[c],
                device_id=(1 - xx, yy),
                device_id_type=pl.DeviceIdType.MESH,
            )
            d2.start()
            rdma2.append(d2)

        for c in range(K):
            rdma2[c].wait_recv()
        for c in range(K):
            rdma2[c].wait_send()

    out = pl.pallas_call(
        body,
        out_shape=jax.ShapeDtypeStruct((T, D), jnp.bfloat16),
        in_specs=[
            pl.BlockSpec(memory_space=pltpu.SMEM),
            pl.BlockSpec(memory_space=pltpu.SMEM),
            pl.BlockSpec(memory_space=pltpu.SMEM),
            pl.BlockSpec(memory_space=pltpu.VMEM),
            pl.BlockSpec(memory_space=pl.ANY),
        ],
        out_specs=pl.BlockSpec(memory_space=pltpu.VMEM),
        scratch_shapes=[
            pltpu.VMEM((CH, D), jnp.float32),
            pltpu.VMEM((HALF, D), jnp.bfloat16),
            pltpu.SemaphoreType.DMA,
            pltpu.SemaphoreType.DMA((K,)),
            pltpu.SemaphoreType.DMA((K,)),
            pltpu.SemaphoreType.DMA((K,)),
            pltpu.SemaphoreType.DMA((K,)),
        ],
        compiler_params=pltpu.CompilerParams(collective_id=0),
    )(rows, order, cum, ids2d, E)
    return out


# device time: 141134 ns/iter; 1.3641x vs baseline; 1.0234x over previous
import jax
import jax.numpy as jnp
from jax import lax
from jax.experimental import pallas as pl
from jax.experimental.pallas import tpu as pltpu

T = 4096
V_SHARD = 8192
D = 2048
HALF = T // 2
K = 8
CH = HALF // K


def kernel(ids, E):
    x = lax.axis_index("x")
    y = lax.axis_index("y")
    seg = lax.dynamic_slice(ids, (x * HALF,), (HALF,))
    lseg = seg - y * V_SHARD
    inr = (lseg >= 0) & (lseg < V_SHARD)
    key = (jnp.where(inr, 0, 1).astype(jnp.int32) * (1 << 24)
           + jnp.arange(HALF, dtype=jnp.int32) * (1 << 13)
           + jnp.clip(lseg, 0, V_SHARD - 1).astype(jnp.int32))
    skey = jnp.sort(key)
    rows = skey & (V_SHARD - 1)
    order = (skey >> 13) & (HALF - 1)
    cum = jnp.cumsum(
        inr.reshape(K, CH).sum(axis=1, dtype=jnp.int32)).astype(jnp.int32)
    ids2d = ids[:, None]

    def body(rows_s, pos_s, cum_s, idv_ref, e_ref, out_ref,
             gstage, recv1, gsem, s1, r1, s2, r2):
        xx = lax.axis_index("x")
        yy = lax.axis_index("y")
        base = xx * HALF
        vlo = yy * V_SHARD

        barrier_sem = pltpu.get_barrier_semaphore()
        pl.semaphore_signal(barrier_sem, inc=1, device_id=(xx, 1 - yy),
                            device_id_type=pl.DeviceIdType.MESH)
        pl.semaphore_signal(barrier_sem, inc=1, device_id=(1 - xx, yy),
                            device_id_type=pl.DeviceIdType.MESH)
        pl.semaphore_wait(barrier_sem, 2)

        rdma1 = []
        rdma2 = []
        for c in range(K):
            lo = base + c * CH
            start = cum_s[c - 1] if c > 0 else 0
            end = cum_s[c]

            def issue(j, _, off=c * CH):
                pltpu.make_async_copy(
                    e_ref.at[pl.ds(rows_s[j], 1), :],
                    gstage.at[pl.ds(pos_s[j] - off, 1), :],
                    gsem,
                ).start()
                return 0

            def drain(j, _):
                pltpu.make_async_copy(
                    e_ref.at[pl.ds(0, 1), :],
                    gstage.at[pl.ds(0, 1), :],
                    gsem,
                ).wait()
                return 0

            lax.fori_loop(start, end, issue, 0)
            lax.fori_loop(start, end, drain, 0)

            out_ref[pl.ds(lo, CH), :] = gstage[:, :].astype(jnp.bfloat16)
            d1 = pltpu.make_async_remote_copy(
                src_ref=out_ref.at[pl.ds(lo, CH), :],
                dst_ref=recv1.at[pl.ds(c * CH, CH), :],
                send_sem=s1.at[c],
                recv_sem=r1.at[c],
                device_id=(xx, 1 - yy),
                device_id_type=pl.DeviceIdType.MESH,
            )
            d1.start()
            rdma1.append(d1)

        for c in range(K):
            lo = base + c * CH
            rdma1[c].wait_recv()
            rdma1[c].wait_send()
            sel = (idv_ref[pl.ds(lo, CH), :] >= vlo) & (
                idv_ref[pl.ds(lo, CH), :] < vlo + V_SHARD)
            out_ref[pl.ds(lo, CH), :] = jnp.where(
                sel, out_ref[pl.ds(lo, CH), :], recv1[pl.ds(c * CH, CH), :])
            d2 = pltpu.make_async_remote_copy(
                src_ref=out_ref.at[pl.ds(lo, CH), :],
                dst_ref=out_ref.at[pl.ds(lo, CH), :],
                send_sem=s2.at[c],
                recv_sem=r2.at[c],
                device_id=(1 - xx, yy),
                device_id_type=pl.DeviceIdType.MESH,
            )
            d2.start()
            rdma2.append(d2)

        for c in range(K):
            rdma2[c].wait_recv()
        for c in range(K):
            rdma2[c].wait_send()

    out = pl.pallas_call(
        body,
        out_shape=jax.ShapeDtypeStruct((T, D), jnp.bfloat16),
        in_specs=[
            pl.BlockSpec(memory_space=pltpu.SMEM),
            pl.BlockSpec(memory_space=pltpu.SMEM),
            pl.BlockSpec(memory_space=pltpu.SMEM),
            pl.BlockSpec(memory_space=pltpu.VMEM),
            pl.BlockSpec(memory_space=pl.ANY),
        ],
        out_specs=pl.BlockSpec(memory_space=pltpu.VMEM),
        scratch_shapes=[
            pltpu.VMEM((CH, D), jnp.float32),
            pltpu.VMEM((HALF, D), jnp.bfloat16),
            pltpu.SemaphoreType.DMA,
            pltpu.SemaphoreType.DMA((K,)),
            pltpu.SemaphoreType.DMA((K,)),
            pltpu.SemaphoreType.DMA((K,)),
            pltpu.SemaphoreType.DMA((K,)),
        ],
        compiler_params=pltpu.CompilerParams(collective_id=0),
    )(rows, order, cum, ids2d, E)
    return out


# device time: 128867 ns/iter; 1.4940x vs baseline; 1.0952x over previous
import jax
import jax.numpy as jnp
from jax import lax
from jax.experimental import pallas as pl
from jax.experimental.pallas import tpu as pltpu

T = 4096
V_SHARD = 8192
D = 2048
HALF = T // 2
K = 8
CH = HALF // K


def kernel(ids, E):
    x = lax.axis_index("x")
    y = lax.axis_index("y")
    seg = lax.dynamic_slice(ids, (x * HALF,), (HALF,))
    lseg = seg - y * V_SHARD
    inr = (lseg >= 0) & (lseg < V_SHARD)
    key = (jnp.where(inr, 0, 1).astype(jnp.int32) * (1 << 24)
           + jnp.arange(HALF, dtype=jnp.int32) * (1 << 13)
           + jnp.clip(lseg, 0, V_SHARD - 1).astype(jnp.int32))
    skey = jnp.sort(key)
    rows = skey & (V_SHARD - 1)
    order = (skey >> 13) & (HALF - 1)
    cum = jnp.cumsum(
        inr.reshape(K, CH).sum(axis=1, dtype=jnp.int32)).astype(jnp.int32)
    ids2d = ids[:, None]

    def body(rows_s, pos_s, cum_s, idv_ref, e_ref, out_ref,
             gstage, recv1, gsem, s1, r1, s2, r2):
        xx = lax.axis_index("x")
        yy = lax.axis_index("y")
        base = xx * HALF
        vlo = yy * V_SHARD

        barrier_sem = pltpu.get_barrier_semaphore()
        pl.semaphore_signal(barrier_sem, inc=1, device_id=(xx, 1 - yy),
                            device_id_type=pl.DeviceIdType.MESH)
        pl.semaphore_signal(barrier_sem, inc=1, device_id=(1 - xx, yy),
                            device_id_type=pl.DeviceIdType.MESH)
        pl.semaphore_wait(barrier_sem, 2)

        rdma1 = []
        rdma2 = []

        def gather_issue(c):
            start = cum_s[c - 1] if c > 0 else 0

            def issue(j, _, off=c * CH, slot=c % 2):
                pltpu.make_async_copy(
                    e_ref.at[pl.ds(rows_s[j], 1), :],
                    gstage.at[slot, pl.ds(pos_s[j] - off, 1), :],
                    gsem.at[c % 2],
                ).start()
                return 0

            lax.fori_loop(start, cum_s[c], issue, 0)

        def gather_drain(c):
            start = cum_s[c - 1] if c > 0 else 0

            def drain(j, _, slot=c % 2):
                pltpu.make_async_copy(
                    e_ref.at[pl.ds(0, 1), :],
                    gstage.at[slot, pl.ds(0, 1), :],
                    gsem.at[c % 2],
                ).wait()
                return 0

            lax.fori_loop(start, cum_s[c], drain, 0)

        def process(c):
            lo = base + c * CH
            rdma1[c].wait_recv()
            rdma1[c].wait_send()
            sel = (idv_ref[pl.ds(lo, CH), :] >= vlo) & (
                idv_ref[pl.ds(lo, CH), :] < vlo + V_SHARD)
            out_ref[pl.ds(lo, CH), :] = jnp.where(
                sel, out_ref[pl.ds(lo, CH), :], recv1[pl.ds(c * CH, CH), :])
            d2 = pltpu.make_async_remote_copy(
                src_ref=out_ref.at[pl.ds(lo, CH), :],
                dst_ref=out_ref.at[pl.ds(lo, CH), :],
                send_sem=s2.at[c],
                recv_sem=r2.at[c],
                device_id=(1 - xx, yy),
                device_id_type=pl.DeviceIdType.MESH,
            )
            d2.start()
            rdma2.append(d2)

        gather_issue(0)
        for c in range(K):
            if c + 1 < K:
                gather_issue(c + 1)
            gather_drain(c)
            lo = base + c * CH
            out_ref[pl.ds(lo, CH), :] = gstage[c % 2].astype(jnp.bfloat16)
            d1 = pltpu.make_async_remote_copy(
                src_ref=out_ref.at[pl.ds(lo, CH), :],
                dst_ref=recv1.at[pl.ds(c * CH, CH), :],
                send_sem=s1.at[c],
                recv_sem=r1.at[c],
                device_id=(xx, 1 - yy),
                device_id_type=pl.DeviceIdType.MESH,
            )
            d1.start()
            rdma1.append(d1)
            if c >= 2:
                process(c - 2)
        process(K - 2)
        process(K - 1)

        for c in range(K):
            rdma2[c].wait_recv()
        for c in range(K):
            rdma2[c].wait_send()

    out = pl.pallas_call(
        body,
        out_shape=jax.ShapeDtypeStruct((T, D), jnp.bfloat16),
        in_specs=[
            pl.BlockSpec(memory_space=pltpu.SMEM),
            pl.BlockSpec(memory_space=pltpu.SMEM),
            pl.BlockSpec(memory_space=pltpu.SMEM),
            pl.BlockSpec(memory_space=pltpu.VMEM),
            pl.BlockSpec(memory_space=pl.ANY),
        ],
        out_specs=pl.BlockSpec(memory_space=pltpu.VMEM),
        scratch_shapes=[
            pltpu.VMEM((2, CH, D), jnp.float32),
            pltpu.VMEM((HALF, D), jnp.bfloat16),
            pltpu.SemaphoreType.DMA((2,)),
            pltpu.SemaphoreType.DMA((K,)),
            pltpu.SemaphoreType.DMA((K,)),
            pltpu.SemaphoreType.DMA((K,)),
            pltpu.SemaphoreType.DMA((K,)),
        ],
        compiler_params=pltpu.CompilerParams(collective_id=0),
    )(rows, order, cum, ids2d, E)
    return out


# device time: 120542 ns/iter; 1.5972x vs baseline; 1.0691x over previous
import jax
import jax.numpy as jnp
from jax import lax
from jax.experimental import pallas as pl
from jax.experimental.pallas import tpu as pltpu

T = 4096
V_SHARD = 8192
D = 2048
HALF = T // 2
K = 16
CH = HALF // K


def kernel(ids, E):
    x = lax.axis_index("x")
    y = lax.axis_index("y")
    seg = lax.dynamic_slice(ids, (x * HALF,), (HALF,))
    lseg = seg - y * V_SHARD
    inr = (lseg >= 0) & (lseg < V_SHARD)
    key = (jnp.where(inr, 0, 1).astype(jnp.int32) * (1 << 24)
           + jnp.arange(HALF, dtype=jnp.int32) * (1 << 13)
           + jnp.clip(lseg, 0, V_SHARD - 1).astype(jnp.int32))
    skey = jnp.sort(key)
    rows = skey & (V_SHARD - 1)
    order = (skey >> 13) & (HALF - 1)
    cum = jnp.cumsum(
        inr.reshape(K, CH).sum(axis=1, dtype=jnp.int32)).astype(jnp.int32)
    ids2d = ids[:, None]

    def body(rows_s, pos_s, cum_s, idv_ref, e_ref, out_ref,
             gstage, recv1, gsem, s1, r1, s2, r2):
        xx = lax.axis_index("x")
        yy = lax.axis_index("y")
        base = xx * HALF
        vlo = yy * V_SHARD

        barrier_sem = pltpu.get_barrier_semaphore()
        pl.semaphore_signal(barrier_sem, inc=1, device_id=(xx, 1 - yy),
                            device_id_type=pl.DeviceIdType.MESH)
        pl.semaphore_signal(barrier_sem, inc=1, device_id=(1 - xx, yy),
                            device_id_type=pl.DeviceIdType.MESH)
        pl.semaphore_wait(barrier_sem, 2)

        rdma1 = []
        rdma2 = []

        def gather_issue(c):
            start = cum_s[c - 1] if c > 0 else 0

            def issue(j, _, off=c * CH, slot=c % 2):
                pltpu.make_async_copy(
                    e_ref.at[pl.ds(rows_s[j], 1), :],
                    gstage.at[slot, pl.ds(pos_s[j] - off, 1), :],
                    gsem.at[c % 2],
                ).start()
                return 0

            lax.fori_loop(start, cum_s[c], issue, 0)

        def gather_drain(c):
            start = cum_s[c - 1] if c > 0 else 0

            def drain(j, _, slot=c % 2):
                pltpu.make_async_copy(
                    e_ref.at[pl.ds(0, 1), :],
                    gstage.at[slot, pl.ds(0, 1), :],
                    gsem.at[c % 2],
                ).wait()
                return 0

            lax.fori_loop(start, cum_s[c], drain, 0)

        def process(c):
            lo = base + c * CH
            rdma1[c].wait_recv()
            rdma1[c].wait_send()
            sel = (idv_ref[pl.ds(lo, CH), :] >= vlo) & (
                idv_ref[pl.ds(lo, CH), :] < vlo + V_SHARD)
            out_ref[pl.ds(lo, CH), :] = jnp.where(
                sel, out_ref[pl.ds(lo, CH), :], recv1[pl.ds(c * CH, CH), :])
            d2 = pltpu.make_async_remote_copy(
                src_ref=out_ref.at[pl.ds(lo, CH), :],
                dst_ref=out_ref.at[pl.ds(lo, CH), :],
                send_sem=s2.at[c],
                recv_sem=r2.at[c],
                device_id=(1 - xx, yy),
                device_id_type=pl.DeviceIdType.MESH,
            )
            d2.start()
            rdma2.append(d2)

        gather_issue(0)
        for c in range(K):
            if c + 1 < K:
                gather_issue(c + 1)
            gather_drain(c)
            lo = base + c * CH
            out_ref[pl.ds(lo, CH), :] = gstage[c % 2].astype(jnp.bfloat16)
            d1 = pltpu.make_async_remote_copy(
                src_ref=out_ref.at[pl.ds(lo, CH), :],
                dst_ref=recv1.at[pl.ds(c * CH, CH), :],
                send_sem=s1.at[c],
                recv_sem=r1.at[c],
                device_id=(xx, 1 - yy),
                device_id_type=pl.DeviceIdType.MESH,
            )
            d1.start()
            rdma1.append(d1)
            if c >= 2:
                process(c - 2)
        process(K - 2)
        process(K - 1)

        for c in range(K):
            rdma2[c].wait_recv()
        for c in range(K):
            rdma2[c].wait_send()

    out = pl.pallas_call(
        body,
        out_shape=jax.ShapeDtypeStruct((T, D), jnp.bfloat16),
        in_specs=[
            pl.BlockSpec(memory_space=pltpu.SMEM),
            pl.BlockSpec(memory_space=pltpu.SMEM),
            pl.BlockSpec(memory_space=pltpu.SMEM),
            pl.BlockSpec(memory_space=pltpu.VMEM),
            pl.BlockSpec(memory_space=pl.ANY),
        ],
        out_specs=pl.BlockSpec(memory_space=pltpu.VMEM),
        scratch_shapes=[
            pltpu.VMEM((2, CH, D), jnp.float32),
            pltpu.VMEM((HALF, D), jnp.bfloat16),
            pltpu.SemaphoreType.DMA((2,)),
            pltpu.SemaphoreType.DMA((K,)),
            pltpu.SemaphoreType.DMA((K,)),
            pltpu.SemaphoreType.DMA((K,)),
            pltpu.SemaphoreType.DMA((K,)),
        ],
        compiler_params=pltpu.CompilerParams(collective_id=0),
    )(rows, order, cum, ids2d, E)
    return out


# device time: 119772 ns/iter; 1.6074x vs baseline; 1.0064x over previous
import jax
import jax.numpy as jnp
from jax import lax
from jax.experimental import pallas as pl
from jax.experimental.pallas import tpu as pltpu

T = 4096
V_SHARD = 8192
D = 2048
HALF = T // 2
K = 16
CH = HALF // K


def kernel(ids, E):
    x = lax.axis_index("x")
    y = lax.axis_index("y")
    seg = lax.dynamic_slice(ids, (x * HALF,), (HALF,))
    lseg = seg - y * V_SHARD
    inr = (lseg >= 0) & (lseg < V_SHARD)
    key = (jnp.where(inr, 0, 1).astype(jnp.int32) * (1 << 24)
           + jnp.arange(HALF, dtype=jnp.int32) * (1 << 13)
           + jnp.clip(lseg, 0, V_SHARD - 1).astype(jnp.int32))
    skey = jnp.sort(key)
    rows = skey & (V_SHARD - 1)
    order = (skey >> 13) & (HALF - 1)
    cum = jnp.cumsum(
        inr.reshape(K, CH).sum(axis=1, dtype=jnp.int32)).astype(jnp.int32)
    ids2d = ids[:, None]

    def body(rows_s, pos_s, cum_s, idv_ref, e_ref, out_ref,
             gstage, recv1, gsem, s1, r1, s2, r2):
        xx = lax.axis_index("x")
        yy = lax.axis_index("y")
        base = xx * HALF
        vlo = yy * V_SHARD

        barrier_sem = pltpu.get_barrier_semaphore()
        pl.semaphore_signal(barrier_sem, inc=1, device_id=(xx, 1 - yy),
                            device_id_type=pl.DeviceIdType.MESH)
        pl.semaphore_signal(barrier_sem, inc=1, device_id=(1 - xx, yy),
                            device_id_type=pl.DeviceIdType.MESH)
        pl.semaphore_wait(barrier_sem, 2)

        rdma1 = []
        rdma2 = []

        U = 4

        def nwaves(c):
            start = cum_s[c - 1] if c > 0 else 0
            return start, cum_s[c], (cum_s[c] - start + U - 1) // U

        def gather_issue(c):
            start, end, nw = nwaves(c)

            def wave(w, _, off=c * CH, slot=c % 2):
                for u in range(U):
                    j = start + w * U + u
                    sj = jnp.minimum(j, HALF - 1)
                    pp = jnp.where(j < end, pos_s[sj] - off, CH)
                    pltpu.make_async_copy(
                        e_ref.at[pl.ds(rows_s[sj], 1), :],
                        gstage.at[slot, pl.ds(pp, 1), :],
                        gsem.at[c % 2],
                    ).start()
                return 0

            lax.fori_loop(0, nw, wave, 0)

        def gather_drain(c):
            _, _, nw = nwaves(c)

            def wave(w, _, slot=c % 2):
                for u in range(U):
                    pltpu.make_async_copy(
                        e_ref.at[pl.ds(0, 1), :],
                        gstage.at[slot, pl.ds(0, 1), :],
                        gsem.at[c % 2],
                    ).wait()
                return 0

            lax.fori_loop(0, nw, wave, 0)

        def process(c):
            lo = base + c * CH
            rdma1[c].wait_recv()
            rdma1[c].wait_send()
            sel = (idv_ref[pl.ds(lo, CH), :] >= vlo) & (
                idv_ref[pl.ds(lo, CH), :] < vlo + V_SHARD)
            out_ref[pl.ds(lo, CH), :] = jnp.where(
                sel, out_ref[pl.ds(lo, CH), :], recv1[pl.ds(c * CH, CH), :])
            d2 = pltpu.make_async_remote_copy(
                src_ref=out_ref.at[pl.ds(lo, CH), :],
                dst_ref=out_ref.at[pl.ds(lo, CH), :],
                send_sem=s2.at[c],
                recv_sem=r2.at[c],
                device_id=(1 - xx, yy),
                device_id_type=pl.DeviceIdType.MESH,
            )
            d2.start()
            rdma2.append(d2)

        gather_issue(0)
        for c in range(K):
            if c + 1 < K:
                gather_issue(c + 1)
            gather_drain(c)
            lo = base + c * CH
            out_ref[pl.ds(lo, CH), :] = gstage[c % 2, :CH, :].astype(jnp.bfloat16)
            d1 = pltpu.make_async_remote_copy(
                src_ref=out_ref.at[pl.ds(lo, CH), :],
                dst_ref=recv1.at[pl.ds(c * CH, CH), :],
                send_sem=s1.at[c],
                recv_sem=r1.at[c],
                device_id=(xx, 1 - yy),
                device_id_type=pl.DeviceIdType.MESH,
            )
            d1.start()
            rdma1.append(d1)
            if c >= 2:
                process(c - 2)
        process(K - 2)
        process(K - 1)

        for c in range(K):
            rdma2[c].wait_recv()
        for c in range(K):
            rdma2[c].wait_send()

    out = pl.pallas_call(
        body,
        out_shape=jax.ShapeDtypeStruct((T, D), jnp.bfloat16),
        in_specs=[
            pl.BlockSpec(memory_space=pltpu.SMEM),
            pl.BlockSpec(memory_space=pltpu.SMEM),
            pl.BlockSpec(memory_space=pltpu.SMEM),
            pl.BlockSpec(memory_space=pltpu.VMEM),
            pl.BlockSpec(memory_space=pl.ANY),
        ],
        out_specs=pl.BlockSpec(memory_space=pltpu.VMEM),
        scratch_shapes=[
            pltpu.VMEM((2, CH + 1, D), jnp.float32),
            pltpu.VMEM((HALF, D), jnp.bfloat16),
            pltpu.SemaphoreType.DMA((2,)),
            pltpu.SemaphoreType.DMA((K,)),
            pltpu.SemaphoreType.DMA((K,)),
            pltpu.SemaphoreType.DMA((K,)),
            pltpu.SemaphoreType.DMA((K,)),
        ],
        compiler_params=pltpu.CompilerParams(collective_id=0),
    )(rows, order, cum, ids2d, E)
    return out
